# baseline (speedup 1.0000x reference)
import functools

import jax
import jax.numpy as jnp
from jax import lax
from jax.experimental import pallas as pl
from jax.experimental.pallas import tpu as pltpu
from jax.experimental.pallas import tpu_sc as plsc

_N = 10000
_E = 320000
_D_IN = 128
_D_HID = 256
_D_OUT = 64

_NC = 2
_NS = 16
_NW = _NC * _NS

_CHUNK = 128
_EPT_ROWS = 80
_E_PAD = _NW * _EPT_ROWS * _CHUNK
_ROWS_PER_TILE_N = 640
_N_PAD = _NS * _ROWS_PER_TILE_N
_CNT_W = 16

def _sc_mesh():
    return plsc.VectorSubcoreMesh(
        core_axis_name="c", subcore_axis_name="s",
        num_cores=_NC, num_subcores=_NS,
    )


def _make_sc_agg(width, chunk):
    rows = _E_PAD // (_NW * chunk)
    out_type = jax.ShapeDtypeStruct((_NC, _N_PAD, width), jnp.float32)
    scratch = [
        pltpu.VMEM((rows, chunk), jnp.int32),
        pltpu.VMEM((rows, chunk), jnp.int32),
        pltpu.VMEM((chunk, width), jnp.float32),
        pltpu.VMEM((chunk, width), jnp.float32),
        pltpu.VMEM_SHARED((_N_PAD, width), jnp.float32),
        pltpu.SemaphoreType.DMA,
        pltpu.SemaphoreType.DMA,
    ]

    def body(table, srcm, dstm, zrow, acc_out, src_v, dst_v, rows0, rows1,
             acc_sh, sem0, sem1):
        cid = lax.axis_index("c")
        sid = lax.axis_index("s")
        wid = sid * _NC + cid
        row0 = sid * _ROWS_PER_TILE_N
        sl = pl.ds(row0, _ROWS_PER_TILE_N)
        bufs = ((rows0, sem0), (rows1, sem1))

        pltpu.sync_copy(zrow, acc_sh.at[sl])
        pltpu.sync_copy(srcm.at[pl.ds(wid * rows, rows)], src_v)
        pltpu.sync_copy(dstm.at[pl.ds(wid * rows, rows)], dst_v)
        plsc.subcore_barrier()

        pltpu.async_copy(table.at[dst_v.at[0]], rows0, sem0)

        def step(k, carry):
            for b in range(2):
                j = 2 * k + b
                buf, sem = bufs[b]
                nbuf, nsem = bufs[1 - b]
                pltpu.make_async_copy(table.at[dst_v.at[j]], buf, sem).wait()
                jn = jnp.minimum(j + 1, rows - 1)
                pltpu.async_copy(table.at[dst_v.at[jn]], nbuf, nsem)
                pltpu.sync_copy(buf, acc_sh.at[src_v.at[j]], add=True)
            return carry

        lax.fori_loop(0, rows // 2, step, 0)
        pltpu.make_async_copy(table.at[dst_v.at[rows - 1]], rows0, sem0).wait()
        plsc.subcore_barrier()
        pltpu.sync_copy(acc_sh.at[sl], acc_out.at[cid, sl])

    return pl.kernel(
        body, out_type=out_type, mesh=_sc_mesh(), scratch_types=scratch,
        compiler_params=pltpu.CompilerParams(use_tc_tiling_on_sc=False))


def _make_sc_agg_fs(width, edge_split=False):
    half = width if edge_split else width // 2
    chunk = _CHUNK
    rows_t = _E_PAD // ((_NW if edge_split else _NS) * chunk)
    n_ph = 4
    ph = rows_t // n_ph
    rpt_tab = _N // _NS
    out_type = jax.ShapeDtypeStruct((_NC, _N_PAD, half), jnp.float32)
    scratch = [
        pltpu.VMEM((ph, chunk), jnp.int32),
        pltpu.VMEM((ph, chunk), jnp.int32),
        [pltpu.VMEM((chunk, half), jnp.float32) for _ in range(4)],
        pltpu.VMEM_SHARED((_N, half), jnp.float32),
        pltpu.VMEM_SHARED((_N_PAD, half), jnp.float32),
        [pltpu.SemaphoreType.DMA for _ in range(4)],
        [pltpu.SemaphoreType.DMA for _ in range(4)],
    ]

    def body(tabh, srcm, dstm, zrow, acc_out, src_v, dst_v, bufs, tab_sh,
             acc_sh, gsems, ssems):
        cid = lax.axis_index("c")
        sid = lax.axis_index("s")
        row0 = sid * _ROWS_PER_TILE_N
        sl = pl.ds(row0, _ROWS_PER_TILE_N)

        def gather(j, b):
            pltpu.async_copy(tab_sh.at[dst_v.at[j]], bufs[b], gsems[b])

        def gather_wait(b):
            pltpu.make_async_copy(tab_sh.at[dst_v.at[0]], bufs[b],
                                  gsems[b]).wait()

        def scatter(j, b):
            pltpu.async_copy(bufs[b], acc_sh.at[src_v.at[j]], ssems[b],
                             add=True)

        def scatter_wait(b):
            pltpu.make_async_copy(bufs[b], acc_sh.at[src_v.at[0]],
                                  ssems[b]).wait()

        tsl = pl.ds(sid * rpt_tab, rpt_tab)
        if edge_split:
            pltpu.sync_copy(tabh.at[tsl], tab_sh.at[tsl])
        else:
            pltpu.sync_copy(tabh.at[cid, tsl], tab_sh.at[tsl])
        pltpu.sync_copy(zrow, acc_sh.at[sl])
        plsc.subcore_barrier()

        wid = sid * _NC + cid
        for p in range(n_ph):
            base = (wid if edge_split else sid) * rows_t + p * ph
            pltpu.sync_copy(srcm.at[pl.ds(base, ph)], src_v)
            pltpu.sync_copy(dstm.at[pl.ds(base, ph)], dst_v)
            for b in range(4):
                gather(b, b)
            for i in range(2):
                gather_wait(i)
                scatter(i, i)

            def step(k, carry):
                for b4 in range(4):
                    i = 4 * k + 2 + b4
                    b = (2 + b4) % 4
                    gather_wait(b)
                    scatter(i, b)
                    bn = b4 % 4
                    scatter_wait(bn)
                    gather(jnp.minimum(i + 2, ph - 1), bn)
                return carry

            lax.fori_loop(0, (ph - 4) // 4, step, 0)
            for i in (ph - 2, ph - 1):
                b = i % 4
                gather_wait(b)
                scatter(i, b)
            for i in range(ph - 4, ph):
                scatter_wait(i % 4)

        plsc.subcore_barrier()
        pltpu.sync_copy(acc_sh.at[sl], acc_out.at[cid, sl])

    return pl.kernel(
        body, out_type=out_type, mesh=_sc_mesh(), scratch_types=scratch,
        compiler_params=pltpu.CompilerParams(use_tc_tiling_on_sc=False))


def _make_sc_counts():
    out_type = jax.ShapeDtypeStruct((_NC, _N_PAD, _CNT_W), jnp.float32)
    scratch = [
        pltpu.VMEM((_EPT_ROWS, _CHUNK), jnp.int32),
        pltpu.VMEM((_CHUNK, _CNT_W), jnp.float32),
        pltpu.VMEM_SHARED((_N_PAD, _CNT_W), jnp.float32),
    ]

    def body(srcm, zcnt, ones_hbm, cnt_out, src_v, ones_v, cnt_sh):
        cid = lax.axis_index("c")
        sid = lax.axis_index("s")
        wid = sid * _NC + cid
        row0 = sid * _ROWS_PER_TILE_N
        sl = pl.ds(row0, _ROWS_PER_TILE_N)

        pltpu.sync_copy(zcnt, cnt_sh.at[sl])
        pltpu.sync_copy(ones_hbm, ones_v)
        pltpu.sync_copy(srcm.at[pl.ds(wid * _EPT_ROWS, _EPT_ROWS)], src_v)
        plsc.subcore_barrier()

        def step(j, carry):
            pltpu.sync_copy(ones_v, cnt_sh.at[src_v.at[j]], add=True)
            return carry

        lax.fori_loop(0, _EPT_ROWS, step, 0)
        plsc.subcore_barrier()
        pltpu.sync_copy(cnt_sh.at[sl], cnt_out.at[cid, sl])

    return pl.kernel(
        body, out_type=out_type, mesh=_sc_mesh(), scratch_types=scratch,
        compiler_params=pltpu.CompilerParams(use_tc_tiling_on_sc=False))


def _tc1_body(acc_ref, cnt_ref, x_ref, w1_ref, w2_ref, b1_ref, g_ref):
    c = cnt_ref[0, :, 0:1] + cnt_ref[1, :, 0:1] + 1.0
    s = jnp.concatenate([acc_ref[0], acc_ref[1]], axis=1) + x_ref[...]
    agg = s / c
    h = lax.dot_general(agg, w1_ref[...], (((1,), (1,)), ((), ())),
                        preferred_element_type=jnp.float32)
    h = jnp.maximum(h + b1_ref[...], 0.0)
    g_ref[...] = lax.dot_general(h, w2_ref[...], (((1,), (1,)), ((), ())),
                                 preferred_element_type=jnp.float32)


def _tc2_body(acc_ref, cnt_ref, g_ref, b2_ref, out_ref):
    c = cnt_ref[0, :, 0:1] + cnt_ref[1, :, 0:1] + 1.0
    z = (acc_ref[0] + acc_ref[1] + g_ref[...]) / c + b2_ref[...]
    m = jnp.max(z, axis=1, keepdims=True)
    lse = jnp.log(jnp.sum(jnp.exp(z - m), axis=1, keepdims=True)) + m
    out_ref[...] = z - lse


_R = 1000


def kernel(x, edge_index, W1, b1, W2, b2):
    src = edge_index[0]
    dst = edge_index[1]
    pad = _E_PAD - _E
    src_p = jnp.concatenate([src, jnp.full((pad,), _N, jnp.int32)])
    dst_p = jnp.concatenate([dst, jnp.zeros((pad,), jnp.int32)])
    srcm = src_p.reshape(_E_PAD // _CHUNK, _CHUNK)
    dstm = dst_p.reshape(_E_PAD // _CHUNK, _CHUNK)
    zrow = jnp.zeros((_ROWS_PER_TILE_N, _D_IN // 2), jnp.float32)
    zrow64 = jnp.zeros((_ROWS_PER_TILE_N, _D_OUT), jnp.float32)
    zcnt = jnp.zeros((_ROWS_PER_TILE_N, _CNT_W), jnp.float32)

    ones = jnp.ones((_CHUNK, _CNT_W), jnp.float32)
    cnt = _make_sc_counts()(srcm, zcnt, ones)
    xh = jnp.stack([x[:, : _D_IN // 2], x[:, _D_IN // 2:]])
    acc1 = _make_sc_agg_fs(_D_IN)(xh, srcm, dstm, zrow)

    grid = (_N // _R,)
    gh = pl.pallas_call(
        _tc1_body,
        grid=grid,
        in_specs=[
            pl.BlockSpec((_NC, _R, _D_IN // 2), lambda i: (0, i, 0)),
            pl.BlockSpec((_NC, _R, _CNT_W), lambda i: (0, i, 0)),
            pl.BlockSpec((_R, _D_IN), lambda i: (i, 0)),
            pl.BlockSpec((_D_HID, _D_IN), lambda i: (0, 0)),
            pl.BlockSpec((_D_OUT, _D_HID), lambda i: (0, 0)),
            pl.BlockSpec((1, _D_HID), lambda i: (0, 0)),
        ],
        out_specs=pl.BlockSpec((_R, _D_OUT), lambda i: (i, 0)),
        out_shape=jax.ShapeDtypeStruct((_N, _D_OUT), jnp.float32),
    )(acc1, cnt, x, W1, W2, b1.reshape(1, _D_HID))

    acc2 = _make_sc_agg_fs(_D_OUT, edge_split=True)(gh, srcm, dstm, zrow64)

    out = pl.pallas_call(
        _tc2_body,
        grid=grid,
        in_specs=[
            pl.BlockSpec((_NC, _R, _D_OUT), lambda i: (0, i, 0)),
            pl.BlockSpec((_NC, _R, _CNT_W), lambda i: (0, i, 0)),
            pl.BlockSpec((_R, _D_OUT), lambda i: (i, 0)),
            pl.BlockSpec((1, _D_OUT), lambda i: (0, 0)),
        ],
        out_specs=pl.BlockSpec((_R, _D_OUT), lambda i: (i, 0)),
        out_shape=jax.ShapeDtypeStruct((_N, _D_OUT), jnp.float32),
    )(acc2, cnt, gh, b2.reshape(1, _D_OUT))

    return out


# speedup vs baseline: 1.0355x; 1.0355x over previous
import functools

import jax
import jax.numpy as jnp
from jax import lax
from jax.experimental import pallas as pl
from jax.experimental.pallas import tpu as pltpu
from jax.experimental.pallas import tpu_sc as plsc

_N = 10000
_E = 320000
_D_IN = 128
_D_HID = 256
_D_OUT = 64

_NC = 2
_NS = 16
_NW = _NC * _NS

_CHUNK = 128
_EPT_ROWS = 80
_E_PAD = _NW * _EPT_ROWS * _CHUNK
_ROWS_PER_TILE_N = 640
_N_PAD = _NS * _ROWS_PER_TILE_N
_CNT_W = 16

def _sc_mesh():
    return plsc.VectorSubcoreMesh(
        core_axis_name="c", subcore_axis_name="s",
        num_cores=_NC, num_subcores=_NS,
    )


def _make_sc_agg(width, chunk):
    rows = _E_PAD // (_NW * chunk)
    out_type = jax.ShapeDtypeStruct((_NC, _N_PAD, width), jnp.float32)
    scratch = [
        pltpu.VMEM((rows, chunk), jnp.int32),
        pltpu.VMEM((rows, chunk), jnp.int32),
        pltpu.VMEM((chunk, width), jnp.float32),
        pltpu.VMEM((chunk, width), jnp.float32),
        pltpu.VMEM_SHARED((_N_PAD, width), jnp.float32),
        pltpu.SemaphoreType.DMA,
        pltpu.SemaphoreType.DMA,
    ]

    def body(table, srcm, dstm, zrow, acc_out, src_v, dst_v, rows0, rows1,
             acc_sh, sem0, sem1):
        cid = lax.axis_index("c")
        sid = lax.axis_index("s")
        wid = sid * _NC + cid
        row0 = sid * _ROWS_PER_TILE_N
        sl = pl.ds(row0, _ROWS_PER_TILE_N)
        bufs = ((rows0, sem0), (rows1, sem1))

        pltpu.sync_copy(zrow, acc_sh.at[sl])
        pltpu.sync_copy(srcm.at[pl.ds(wid * rows, rows)], src_v)
        pltpu.sync_copy(dstm.at[pl.ds(wid * rows, rows)], dst_v)
        plsc.subcore_barrier()

        pltpu.async_copy(table.at[dst_v.at[0]], rows0, sem0)

        def step(k, carry):
            for b in range(2):
                j = 2 * k + b
                buf, sem = bufs[b]
                nbuf, nsem = bufs[1 - b]
                pltpu.make_async_copy(table.at[dst_v.at[j]], buf, sem).wait()
                jn = jnp.minimum(j + 1, rows - 1)
                pltpu.async_copy(table.at[dst_v.at[jn]], nbuf, nsem)
                pltpu.sync_copy(buf, acc_sh.at[src_v.at[j]], add=True)
            return carry

        lax.fori_loop(0, rows // 2, step, 0)
        pltpu.make_async_copy(table.at[dst_v.at[rows - 1]], rows0, sem0).wait()
        plsc.subcore_barrier()
        pltpu.sync_copy(acc_sh.at[sl], acc_out.at[cid, sl])

    return pl.kernel(
        body, out_type=out_type, mesh=_sc_mesh(), scratch_types=scratch,
        compiler_params=pltpu.CompilerParams(use_tc_tiling_on_sc=False))


def _make_sc_agg_fs(width, edge_split=False):
    half = width if edge_split else width // 2
    chunk = _CHUNK
    rows_t = _E_PAD // ((_NW if edge_split else _NS) * chunk)
    n_ph = 4
    ph = rows_t // n_ph
    rpt_tab = _N // _NS
    out_type = jax.ShapeDtypeStruct((_NC, _N_PAD, half), jnp.float32)
    scratch = [
        pltpu.VMEM((ph, chunk), jnp.int32),
        pltpu.VMEM((ph, chunk), jnp.int32),
        [pltpu.VMEM((chunk, half), jnp.float32) for _ in range(4)],
        pltpu.VMEM_SHARED((_N, half), jnp.float32),
        pltpu.VMEM_SHARED((_N_PAD, half), jnp.float32),
        [pltpu.SemaphoreType.DMA for _ in range(4)],
        [pltpu.SemaphoreType.DMA for _ in range(4)],
    ]

    def body(tabh, srcm, dstm, zrow, acc_out, src_v, dst_v, bufs, tab_sh,
             acc_sh, gsems, ssems):
        cid = lax.axis_index("c")
        sid = lax.axis_index("s")
        row0 = sid * _ROWS_PER_TILE_N
        sl = pl.ds(row0, _ROWS_PER_TILE_N)

        def gather(j, b):
            pltpu.async_copy(tab_sh.at[dst_v.at[j]], bufs[b], gsems[b])

        def gather_wait(b):
            pltpu.make_async_copy(tab_sh.at[dst_v.at[0]], bufs[b],
                                  gsems[b]).wait()

        def scatter(j, b):
            pltpu.async_copy(bufs[b], acc_sh.at[src_v.at[j]], ssems[b],
                             add=True)

        def scatter_wait(b):
            pltpu.make_async_copy(bufs[b], acc_sh.at[src_v.at[0]],
                                  ssems[b]).wait()

        tsl = pl.ds(sid * rpt_tab, rpt_tab)
        if edge_split:
            pltpu.sync_copy(tabh.at[tsl], tab_sh.at[tsl])
        else:
            pltpu.sync_copy(tabh.at[cid, tsl], tab_sh.at[tsl])
        pltpu.sync_copy(zrow, acc_sh.at[sl])
        plsc.subcore_barrier()

        wid = sid * _NC + cid
        for p in range(n_ph):
            base = (wid if edge_split else sid) * rows_t + p * ph
            pltpu.sync_copy(srcm.at[pl.ds(base, ph)], src_v)
            pltpu.sync_copy(dstm.at[pl.ds(base, ph)], dst_v)
            for b in range(4):
                gather(b, b)
            for i in range(2):
                gather_wait(i)
                scatter(i, i)

            def step(k, carry):
                for b4 in range(4):
                    i = 4 * k + 2 + b4
                    b = (2 + b4) % 4
                    gather_wait(b)
                    scatter(i, b)
                    bn = b4 % 4
                    scatter_wait(bn)
                    gather(jnp.minimum(i + 2, ph - 1), bn)
                return carry

            lax.fori_loop(0, (ph - 4) // 4, step, 0)
            for i in (ph - 2, ph - 1):
                b = i % 4
                gather_wait(b)
                scatter(i, b)
            for i in range(ph - 4, ph):
                scatter_wait(i % 4)

        plsc.subcore_barrier()
        pltpu.sync_copy(acc_sh.at[sl], acc_out.at[cid, sl])

    return pl.kernel(
        body, out_type=out_type, mesh=_sc_mesh(), scratch_types=scratch,
        compiler_params=pltpu.CompilerParams(use_tc_tiling_on_sc=False))


def _make_sc_counts():
    out_type = jax.ShapeDtypeStruct((_NC, _N_PAD, _CNT_W), jnp.float32)
    scratch = [
        pltpu.VMEM((_EPT_ROWS, _CHUNK), jnp.int32),
        pltpu.VMEM((_CHUNK, _CNT_W), jnp.float32),
        pltpu.VMEM_SHARED((_N_PAD, _CNT_W), jnp.float32),
    ]

    def body(srcm, zcnt, ones_hbm, cnt_out, src_v, ones_v, cnt_sh):
        cid = lax.axis_index("c")
        sid = lax.axis_index("s")
        wid = sid * _NC + cid
        row0 = sid * _ROWS_PER_TILE_N
        sl = pl.ds(row0, _ROWS_PER_TILE_N)

        pltpu.sync_copy(zcnt, cnt_sh.at[sl])
        pltpu.sync_copy(ones_hbm, ones_v)
        pltpu.sync_copy(srcm.at[pl.ds(wid * _EPT_ROWS, _EPT_ROWS)], src_v)
        plsc.subcore_barrier()

        def step(j, carry):
            pltpu.sync_copy(ones_v, cnt_sh.at[src_v.at[j]], add=True)
            return carry

        lax.fori_loop(0, _EPT_ROWS, step, 0)
        plsc.subcore_barrier()
        pltpu.sync_copy(cnt_sh.at[sl], cnt_out.at[cid, sl])

    return pl.kernel(
        body, out_type=out_type, mesh=_sc_mesh(), scratch_types=scratch,
        compiler_params=pltpu.CompilerParams(use_tc_tiling_on_sc=False))


def _tc1_body(acc_ref, cnt_ref, x_ref, w1_ref, w2_ref, b1_ref, g_ref):
    c = cnt_ref[0, :, 0:1] + cnt_ref[1, :, 0:1] + 1.0
    s = jnp.concatenate([acc_ref[0], acc_ref[1]], axis=1) + x_ref[...]
    agg = s / c
    h = lax.dot_general(agg, w1_ref[...], (((1,), (1,)), ((), ())),
                        preferred_element_type=jnp.float32)
    h = jnp.maximum(h + b1_ref[...], 0.0)
    gg = lax.dot_general(h, w2_ref[...], (((1,), (1,)), ((), ())),
                         preferred_element_type=jnp.float32)
    g_ref[0] = gg[:, : _D_OUT // 2]
    g_ref[1] = gg[:, _D_OUT // 2:]


def _tc2_body(acc_ref, cnt_ref, g_ref, b2_ref, out_ref):
    c = cnt_ref[0, :, 0:1] + cnt_ref[1, :, 0:1] + 1.0
    s = jnp.concatenate([acc_ref[0], acc_ref[1]], axis=1)
    g = jnp.concatenate([g_ref[0], g_ref[1]], axis=1)
    z = (s + g) / c + b2_ref[...]
    m = jnp.max(z, axis=1, keepdims=True)
    lse = jnp.log(jnp.sum(jnp.exp(z - m), axis=1, keepdims=True)) + m
    out_ref[...] = z - lse


_R = 1000


def kernel(x, edge_index, W1, b1, W2, b2):
    src = edge_index[0]
    dst = edge_index[1]
    pad = _E_PAD - _E
    src_p = jnp.concatenate([src, jnp.full((pad,), _N, jnp.int32)])
    dst_p = jnp.concatenate([dst, jnp.zeros((pad,), jnp.int32)])
    srcm = src_p.reshape(_E_PAD // _CHUNK, _CHUNK)
    dstm = dst_p.reshape(_E_PAD // _CHUNK, _CHUNK)
    zrow = jnp.zeros((_ROWS_PER_TILE_N, _D_IN // 2), jnp.float32)
    zrow64 = jnp.zeros((_ROWS_PER_TILE_N, _D_OUT // 2), jnp.float32)
    zcnt = jnp.zeros((_ROWS_PER_TILE_N, _CNT_W), jnp.float32)

    ones = jnp.ones((_CHUNK, _CNT_W), jnp.float32)
    cnt = _make_sc_counts()(srcm, zcnt, ones)
    xh = jnp.stack([x[:, : _D_IN // 2], x[:, _D_IN // 2:]])
    acc1 = _make_sc_agg_fs(_D_IN)(xh, srcm, dstm, zrow)

    grid = (_N // _R,)
    gh = pl.pallas_call(
        _tc1_body,
        grid=grid,
        in_specs=[
            pl.BlockSpec((_NC, _R, _D_IN // 2), lambda i: (0, i, 0)),
            pl.BlockSpec((_NC, _R, _CNT_W), lambda i: (0, i, 0)),
            pl.BlockSpec((_R, _D_IN), lambda i: (i, 0)),
            pl.BlockSpec((_D_HID, _D_IN), lambda i: (0, 0)),
            pl.BlockSpec((_D_OUT, _D_HID), lambda i: (0, 0)),
            pl.BlockSpec((1, _D_HID), lambda i: (0, 0)),
        ],
        out_specs=pl.BlockSpec((_NC, _R, _D_OUT // 2), lambda i: (0, i, 0)),
        out_shape=jax.ShapeDtypeStruct((_NC, _N, _D_OUT // 2), jnp.float32),
    )(acc1, cnt, x, W1, W2, b1.reshape(1, _D_HID))

    acc2 = _make_sc_agg_fs(_D_OUT)(gh, srcm, dstm, zrow64)

    out = pl.pallas_call(
        _tc2_body,
        grid=grid,
        in_specs=[
            pl.BlockSpec((_NC, _R, _D_OUT // 2), lambda i: (0, i, 0)),
            pl.BlockSpec((_NC, _R, _CNT_W), lambda i: (0, i, 0)),
            pl.BlockSpec((_NC, _R, _D_OUT // 2), lambda i: (0, i, 0)),
            pl.BlockSpec((1, _D_OUT), lambda i: (0, 0)),
        ],
        out_specs=pl.BlockSpec((_R, _D_OUT), lambda i: (i, 0)),
        out_shape=jax.ShapeDtypeStruct((_N, _D_OUT), jnp.float32),
    )(acc2, cnt, gh, b2.reshape(1, _D_OUT))

    return out


# speedup vs baseline: 1.0381x; 1.0026x over previous
import jax
import jax.numpy as jnp
from jax import lax
from jax.experimental import pallas as pl
from jax.experimental.pallas import tpu as pltpu
from jax.experimental.pallas import tpu_sc as plsc

_N = 10000
_E = 320000
_D_IN = 128
_D_HID = 256
_D_OUT = 64

_NC = 2
_NS = 16
_NW = _NC * _NS

_CHUNK = 128
_EPT_ROWS = 80
_E_PAD = _NW * _EPT_ROWS * _CHUNK
_ROWS_PER_TILE_N = 640
_N_PAD = _NS * _ROWS_PER_TILE_N
_CNT_W = 16

def _sc_mesh():
    return plsc.VectorSubcoreMesh(
        core_axis_name="c", subcore_axis_name="s",
        num_cores=_NC, num_subcores=_NS,
    )


def _make_sc_agg_fs(width, edge_split=False):
    half = width if edge_split else width // 2
    chunk = _CHUNK
    rows_t = _E_PAD // ((_NW if edge_split else _NS) * chunk)
    n_ph = 4
    ph = rows_t // n_ph
    rpt_tab = _N // _NS
    out_type = jax.ShapeDtypeStruct((_NC, _N_PAD, half), jnp.float32)
    scratch = [
        pltpu.VMEM((ph, chunk), jnp.int32),
        pltpu.VMEM((ph, chunk), jnp.int32),
        [pltpu.VMEM((chunk, half), jnp.float32) for _ in range(4)],
        pltpu.VMEM_SHARED((_N, half), jnp.float32),
        pltpu.VMEM_SHARED((_N_PAD, half), jnp.float32),
        [pltpu.SemaphoreType.DMA for _ in range(4)],
        [pltpu.SemaphoreType.DMA for _ in range(4)],
    ]

    def body(tabh, srcm, dstm, zrow, acc_out, src_v, dst_v, bufs, tab_sh,
             acc_sh, gsems, ssems):
        cid = lax.axis_index("c")
        sid = lax.axis_index("s")
        row0 = sid * _ROWS_PER_TILE_N
        sl = pl.ds(row0, _ROWS_PER_TILE_N)

        def gather(j, b):
            pltpu.async_copy(tab_sh.at[dst_v.at[j]], bufs[b], gsems[b])

        def gather_wait(b):
            pltpu.make_async_copy(tab_sh.at[dst_v.at[0]], bufs[b],
                                  gsems[b]).wait()

        def scatter(j, b):
            pltpu.async_copy(bufs[b], acc_sh.at[src_v.at[j]], ssems[b],
                             add=True)

        def scatter_wait(b):
            pltpu.make_async_copy(bufs[b], acc_sh.at[src_v.at[0]],
                                  ssems[b]).wait()

        tsl = pl.ds(sid * rpt_tab, rpt_tab)
        if edge_split:
            pltpu.sync_copy(tabh.at[tsl], tab_sh.at[tsl])
        else:
            pltpu.sync_copy(tabh.at[cid, tsl], tab_sh.at[tsl])
        pltpu.sync_copy(zrow, acc_sh.at[sl])
        plsc.subcore_barrier()

        wid = sid * _NC + cid
        for p in range(n_ph):
            base = (wid if edge_split else sid) * rows_t + p * ph
            pltpu.sync_copy(srcm.at[pl.ds(base, ph)], src_v)
            pltpu.sync_copy(dstm.at[pl.ds(base, ph)], dst_v)
            for b in range(4):
                gather(b, b)
            for i in range(2):
                gather_wait(i)
                scatter(i, i)

            def step(k, carry):
                for b4 in range(4):
                    i = 4 * k + 2 + b4
                    b = (2 + b4) % 4
                    gather_wait(b)
                    scatter(i, b)
                    bn = b4 % 4
                    scatter_wait(bn)
                    gather(jnp.minimum(i + 2, ph - 1), bn)
                return carry

            lax.fori_loop(0, (ph - 4) // 4, step, 0)
            for i in (ph - 2, ph - 1):
                b = i % 4
                gather_wait(b)
                scatter(i, b)
            for i in range(ph - 4, ph):
                scatter_wait(i % 4)

        plsc.subcore_barrier()
        pltpu.sync_copy(acc_sh.at[sl], acc_out.at[cid, sl])

    return pl.kernel(
        body, out_type=out_type, mesh=_sc_mesh(), scratch_types=scratch,
        compiler_params=pltpu.CompilerParams(use_tc_tiling_on_sc=False))


def _make_sc_counts():
    out_type = jax.ShapeDtypeStruct((_NC, _N_PAD, _CNT_W), jnp.float32)
    scratch = [
        pltpu.VMEM((_EPT_ROWS, _CHUNK), jnp.int32),
        pltpu.VMEM((_CHUNK, _CNT_W), jnp.float32),
        pltpu.VMEM_SHARED((_N_PAD, _CNT_W), jnp.float32),
    ]

    def body(srcm, zcnt, ones_hbm, cnt_out, src_v, ones_v, cnt_sh):
        cid = lax.axis_index("c")
        sid = lax.axis_index("s")
        wid = sid * _NC + cid
        row0 = sid * _ROWS_PER_TILE_N
        sl = pl.ds(row0, _ROWS_PER_TILE_N)

        pltpu.sync_copy(zcnt, cnt_sh.at[sl])
        pltpu.sync_copy(ones_hbm, ones_v)
        pltpu.sync_copy(srcm.at[pl.ds(wid * _EPT_ROWS, _EPT_ROWS)], src_v)
        plsc.subcore_barrier()

        def step(j, carry):
            pltpu.sync_copy(ones_v, cnt_sh.at[src_v.at[j]], add=True)
            return carry

        lax.fori_loop(0, _EPT_ROWS, step, 0)
        plsc.subcore_barrier()
        pltpu.sync_copy(cnt_sh.at[sl], cnt_out.at[cid, sl])

    return pl.kernel(
        body, out_type=out_type, mesh=_sc_mesh(), scratch_types=scratch,
        compiler_params=pltpu.CompilerParams(use_tc_tiling_on_sc=False))


def _tc1_body(acc_ref, cnt_ref, x_ref, w1_ref, w2_ref, b1_ref, g_ref):
    c = cnt_ref[0, :, 0:1] + cnt_ref[1, :, 0:1] + 1.0
    s = jnp.concatenate([acc_ref[0], acc_ref[1]], axis=1) + x_ref[...]
    agg = s / c
    h = lax.dot_general(agg, w1_ref[...], (((1,), (1,)), ((), ())),
                        preferred_element_type=jnp.float32)
    h = jnp.maximum(h + b1_ref[...], 0.0)
    gg = lax.dot_general(h, w2_ref[...], (((1,), (1,)), ((), ())),
                         preferred_element_type=jnp.float32)
    g_ref[0] = gg[:, : _D_OUT // 2]
    g_ref[1] = gg[:, _D_OUT // 2:]


def _tc2_body(acc_ref, cnt_ref, g_ref, b2_ref, out_ref):
    c = cnt_ref[0, :, 0:1] + cnt_ref[1, :, 0:1] + 1.0
    s = jnp.concatenate([acc_ref[0], acc_ref[1]], axis=1)
    g = jnp.concatenate([g_ref[0], g_ref[1]], axis=1)
    z = (s + g) / c + b2_ref[...]
    m = jnp.max(z, axis=1, keepdims=True)
    lse = jnp.log(jnp.sum(jnp.exp(z - m), axis=1, keepdims=True)) + m
    out_ref[...] = z - lse


_R = 1000


def kernel(x, edge_index, W1, b1, W2, b2):
    src = edge_index[0]
    dst = edge_index[1]
    pad = _E_PAD - _E
    src_p = jnp.concatenate([src, jnp.full((pad,), _N, jnp.int32)])
    dst_p = jnp.concatenate([dst, jnp.zeros((pad,), jnp.int32)])
    srcm = src_p.reshape(_E_PAD // _CHUNK, _CHUNK)
    dstm = dst_p.reshape(_E_PAD // _CHUNK, _CHUNK)
    zrow = jnp.zeros((_ROWS_PER_TILE_N, _D_IN // 2), jnp.float32)
    zrow64 = jnp.zeros((_ROWS_PER_TILE_N, _D_OUT // 2), jnp.float32)
    zcnt = jnp.zeros((_ROWS_PER_TILE_N, _CNT_W), jnp.float32)

    ones = jnp.ones((_CHUNK, _CNT_W), jnp.float32)
    cnt = _make_sc_counts()(srcm, zcnt, ones)
    xh = jnp.stack([x[:, : _D_IN // 2], x[:, _D_IN // 2:]])
    acc1 = _make_sc_agg_fs(_D_IN)(xh, srcm, dstm, zrow)

    grid = (_N // _R,)
    gh = pl.pallas_call(
        _tc1_body,
        grid=grid,
        in_specs=[
            pl.BlockSpec((_NC, _R, _D_IN // 2), lambda i: (0, i, 0)),
            pl.BlockSpec((_NC, _R, _CNT_W), lambda i: (0, i, 0)),
            pl.BlockSpec((_R, _D_IN), lambda i: (i, 0)),
            pl.BlockSpec((_D_HID, _D_IN), lambda i: (0, 0)),
            pl.BlockSpec((_D_OUT, _D_HID), lambda i: (0, 0)),
            pl.BlockSpec((1, _D_HID), lambda i: (0, 0)),
        ],
        out_specs=pl.BlockSpec((_NC, _R, _D_OUT // 2), lambda i: (0, i, 0)),
        out_shape=jax.ShapeDtypeStruct((_NC, _N, _D_OUT // 2), jnp.float32),
    )(acc1, cnt, x, W1, W2, b1.reshape(1, _D_HID))

    acc2 = _make_sc_agg_fs(_D_OUT)(gh, srcm, dstm, zrow64)

    out = pl.pallas_call(
        _tc2_body,
        grid=grid,
        in_specs=[
            pl.BlockSpec((_NC, _R, _D_OUT // 2), lambda i: (0, i, 0)),
            pl.BlockSpec((_NC, _R, _CNT_W), lambda i: (0, i, 0)),
            pl.BlockSpec((_NC, _R, _D_OUT // 2), lambda i: (0, i, 0)),
            pl.BlockSpec((1, _D_OUT), lambda i: (0, 0)),
        ],
        out_specs=pl.BlockSpec((_R, _D_OUT), lambda i: (i, 0)),
        out_shape=jax.ShapeDtypeStruct((_N, _D_OUT), jnp.float32),
    )(acc2, cnt, gh, b2.reshape(1, _D_OUT))

    return out
